# baseline (speedup 1.0000x reference)
import functools

import jax
import jax.numpy as jnp
import numpy as np
from jax import lax
from jax.experimental import pallas as pl
from jax.experimental.pallas import tpu as pltpu
from jax.experimental.pallas import tpu_sc as plsc

_STRIDES = 2
_H = 112
_W = 112
_C = 96
_B = 4
_HD = _H * _STRIDES
_WD = _W * _STRIDES
_NW = 32
_ROWS_PER_W = (_B * _HD) // _NW
_WPB = _HD // _ROWS_PER_W
_WG = _W // 16
_WDG = _WD // 16


def _interp_tables(src_size):
    s = float(src_size - 1)
    d = np.arange(2 * src_size, dtype=np.float64)
    low = (d - 1.0) / 1.5
    high = (d - 1.0 + 0.5 - (s - 1.0) * 2.0) / 1.5 + (s - 1.0)
    mid = (d - 1.0 + 0.5) / 2.0
    src = np.where(d < 2.5, low, np.where(d > 1.0 + (s - 1.0) * 2.0 - 0.5, high, mid))
    r0 = np.floor(src).astype(np.int64)
    fr = src - r0
    w0 = (1.0 - fr) * ((r0 >= 0) & (r0 < src_size))
    w1 = fr * ((r0 + 1 >= 0) & (r0 + 1 < src_size))
    base = np.clip(r0, 0, src_size - 2)
    tap0 = np.select([r0 < 0, r0 > src_size - 2], [w1, 0.0], w0)
    tap1 = np.select([r0 < 0, r0 > src_size - 2], [0.0, w0], w1)
    return (base.astype(np.int32), tap0.astype(np.float32),
            tap1.astype(np.float32))


_R0_NP, _W0_NP, _W1_NP = _interp_tables(_H)
_C0_NP, _V0_NP, _V1_NP = _interp_tables(_W)

_ROW_SPECIALS = [(i, int(_R0_NP[i]), float(_W0_NP[i]), float(_W1_NP[i]))
                 for i in (0, 1, 2, _HD - 3, _HD - 2, _HD - 1)]
_COL_SPECIALS = [(j, int(_C0_NP[j]), float(_V0_NP[j]), float(_V1_NP[j]))
                 for j in (0, 1, 2, _WD - 3, _WD - 2, _WD - 1)]

_PAIRS = _ROWS_PER_W // 2


@functools.partial(
    pl.kernel,
    mesh=plsc.VectorSubcoreMesh(core_axis_name="c", subcore_axis_name="s"),
    out_type=jax.ShapeDtypeStruct((_B, _HD, _C, _WD), jnp.float32),
    compiler_params=pltpu.CompilerParams(
        needs_layout_passes=False, skip_device_barrier=True),
    scratch_types=[
        pltpu.VMEM((_C, _W), jnp.float32),
        pltpu.VMEM((_C, _W), jnp.float32),
        pltpu.VMEM((_C, _W), jnp.float32),
        pltpu.VMEM((_C, _W), jnp.float32),
        pltpu.VMEM((_C * _W,), jnp.float32),
        pltpu.VMEM((_C, _WD), jnp.float32),
        pltpu.VMEM((_C, _WD), jnp.float32),
        pltpu.SemaphoreType.DMA,
        pltpu.SemaphoreType.DMA,
        pltpu.SemaphoreType.DMA,
        pltpu.SemaphoreType.DMA,
    ],
)
def _upsample_sc(x_hbm, out_hbm, a0, b0, a1, b1, row_t, o0, o1,
                 in_sem0, in_sem1, out_sem0, out_sem1):
    cid = lax.axis_index("c")
    sid = lax.axis_index("s")
    wid = sid * 2 + cid
    batch = wid // _WPB
    i_base = (wid % _WPB) * _ROWS_PER_W

    def row_params(i):
        odd = i & 1
        k = i >> 1
        r0 = k - 1 + odd
        w0 = jnp.where(odd == 1, jnp.float32(0.75), jnp.float32(0.25))
        w1 = jnp.where(odd == 1, jnp.float32(0.25), jnp.float32(0.75))
        for si, sr0, sw0, sw1 in _ROW_SPECIALS:
            hit = i == si
            r0 = jnp.where(hit, sr0, r0)
            w0 = jnp.where(hit, jnp.float32(sw0), w0)
            w1 = jnp.where(hit, jnp.float32(sw1), w1)
        return r0, w0, w1

    def fetch(i, a, b, sem):
        r0, _, _ = row_params(i)
        pltpu.async_copy(x_hbm.at[batch, r0], a, sem)
        pltpu.async_copy(x_hbm.at[batch, r0 + 1], b, sem)

    fetch(i_base, a0, b0, in_sem0)
    fetch(i_base + 1, a1, b1, in_sem1)

    jl = lax.iota(jnp.int32, 16)
    zero_v = jl & 0
    fifteen_v = zero_v + 15
    lane0 = jl == 0
    lane15 = jl == 15
    rel_e0 = jnp.maximum((jl - 1) >> 1, 0)
    rel_e1 = (jl + 1) >> 1
    rel_o0 = ((jl + 1) >> 1) + 7
    rel_o1c = jnp.minimum(rel_o0 + 1, 15)
    odd_l = (jl & 1) == 1
    v0_int = jnp.where(odd_l, jnp.float32(0.75), jnp.float32(0.25))
    v1_int = jnp.where(odd_l, jnp.float32(0.25), jnp.float32(0.75))
    v0_b0, v1_b0 = v0_int, v1_int
    v0_b13, v1_b13 = v0_int, v1_int
    idx0_b13 = rel_o0
    for sj, sc0, sv0, sv1 in _COL_SPECIALS:
        if sj < 16:
            hit = jl == sj
            v0_b0 = jnp.where(hit, jnp.float32(sv0), v0_b0)
            v1_b0 = jnp.where(hit, jnp.float32(sv1), v1_b0)
        else:
            hit = jl == (sj - (_WD - 16))
            v0_b13 = jnp.where(hit, jnp.float32(sv0), v0_b13)
            v1_b13 = jnp.where(hit, jnp.float32(sv1), v1_b13)
            idx0_b13 = jnp.where(hit, sc0 - 16 * (_WG - 1), idx0_b13)

    def _dg(vec, idx):
        return jnp.take_along_axis(vec, idx, axis=0)

    _MEM_BLOCKS = (1, 2, 3, 5, 7, 9)

    def do_row(i, m, a, b, o, in_sem, out_sem):
        pltpu.make_async_copy(x_hbm.at[0, 0], a, in_sem).wait()
        pltpu.make_async_copy(x_hbm.at[0, 0], b, in_sem).wait()
        _, w0, w1 = row_params(i)

        @pl.when(m >= 1)
        def _():
            pltpu.make_async_copy(o, out_hbm.at[0, 0], out_sem).wait()

        @plsc.parallel_loop(0, _C, unroll=2)
        def fused(c):
            base = c * _W
            t = []
            for g in range(_WG):
                sl = pl.ds(g * 16, 16)
                tg = w0 * a[c, sl] + w1 * b[c, sl]
                t.append(tg)
                if g < _WG - 1:
                    row_t[pl.ds(base + g * 16, 16)] = tg
            s0 = rel_o0 + base
            se1 = rel_e1 + base
            for blk in range(_WDG):
                g = blk // 2
                if blk == 0:
                    t0 = _dg(t[0], rel_e0)
                    t1 = _dg(t[0], rel_e1)
                    v0, v1 = v0_b0, v1_b0
                elif blk == _WDG - 1:
                    t0 = _dg(t[_WG - 1], idx0_b13)
                    t1 = _dg(t[_WG - 1], rel_o1c)
                    v0, v1 = v0_b13, v1_b13
                elif blk in _MEM_BLOCKS:
                    if blk % 2 == 1:
                        t0 = plsc.load_gather(row_t, [s0 + 16 * g])
                        t1 = plsc.load_gather(row_t, [s0 + (16 * g + 1)])
                    else:
                        t0 = plsc.load_gather(row_t, [se1 + (16 * g - 1)])
                        t1 = plsc.load_gather(row_t, [se1 + 16 * g])
                    v0, v1 = v0_int, v1_int
                elif blk % 2 == 0:
                    t0 = jnp.where(lane0, _dg(t[g - 1], fifteen_v),
                                   _dg(t[g], rel_e0))
                    t1 = _dg(t[g], rel_e1)
                    v0, v1 = v0_int, v1_int
                else:
                    t0 = _dg(t[g], rel_o0)
                    t1 = jnp.where(lane15, _dg(t[g + 1], zero_v),
                                   _dg(t[g], rel_o1c))
                    v0, v1 = v0_int, v1_int
                o[c, pl.ds(blk * 16, 16)] = v0 * t0 + v1 * t1

        pltpu.async_copy(o, out_hbm.at[batch, i], out_sem)

        @pl.when(m < _PAIRS - 1)
        def _():
            fetch(i + 2, a, b, in_sem)

    def per_pair(m, carry):
        i0 = i_base + 2 * m
        do_row(i0, m, a0, b0, o0, in_sem0, out_sem0)
        do_row(i0 + 1, m, a1, b1, o1, in_sem1, out_sem1)
        return carry

    lax.fori_loop(0, _PAIRS, per_pair, 0)
    pltpu.make_async_copy(o0, out_hbm.at[0, 0], out_sem0).wait()
    pltpu.make_async_copy(o1, out_hbm.at[0, 1], out_sem1).wait()


def kernel(inputs):
    x_t = jnp.transpose(inputs, (0, 1, 3, 2))
    out_t = _upsample_sc(x_t)
    return jnp.transpose(out_t, (0, 1, 3, 2))


# speedup vs baseline: 1.0942x; 1.0942x over previous
import functools

import jax
import jax.numpy as jnp
import numpy as np
from jax import lax
from jax.experimental import pallas as pl
from jax.experimental.pallas import tpu as pltpu
from jax.experimental.pallas import tpu_sc as plsc

_STRIDES = 2
_H = 112
_W = 112
_C = 96
_B = 4
_HD = _H * _STRIDES
_WD = _W * _STRIDES
_NW = 32
_ROWS_PER_W = (_B * _HD) // _NW
_WPB = _HD // _ROWS_PER_W
_WG = _W // 16
_WDG = _WD // 16


def _interp_tables(src_size):
    s = float(src_size - 1)
    d = np.arange(2 * src_size, dtype=np.float64)
    low = (d - 1.0) / 1.5
    high = (d - 1.0 + 0.5 - (s - 1.0) * 2.0) / 1.5 + (s - 1.0)
    mid = (d - 1.0 + 0.5) / 2.0
    src = np.where(d < 2.5, low, np.where(d > 1.0 + (s - 1.0) * 2.0 - 0.5, high, mid))
    r0 = np.floor(src).astype(np.int64)
    fr = src - r0
    w0 = (1.0 - fr) * ((r0 >= 0) & (r0 < src_size))
    w1 = fr * ((r0 + 1 >= 0) & (r0 + 1 < src_size))
    base = np.clip(r0, 0, src_size - 2)
    tap0 = np.select([r0 < 0, r0 > src_size - 2], [w1, 0.0], w0)
    tap1 = np.select([r0 < 0, r0 > src_size - 2], [0.0, w0], w1)
    return (base.astype(np.int32), tap0.astype(np.float32),
            tap1.astype(np.float32))


_R0_NP, _W0_NP, _W1_NP = _interp_tables(_H)
_C0_NP, _V0_NP, _V1_NP = _interp_tables(_W)

_ROW_SPECIALS = [(i, int(_R0_NP[i]), float(_W0_NP[i]), float(_W1_NP[i]))
                 for i in (0, 1, 2, _HD - 3, _HD - 2, _HD - 1)]
_COL_SPECIALS = [(j, int(_C0_NP[j]), float(_V0_NP[j]), float(_V1_NP[j]))
                 for j in (0, 1, 2, _WD - 3, _WD - 2, _WD - 1)]

_PAIRS = _ROWS_PER_W // 2


@functools.partial(
    pl.kernel,
    mesh=plsc.VectorSubcoreMesh(core_axis_name="c", subcore_axis_name="s"),
    out_type=jax.ShapeDtypeStruct((_B, _HD, _C, _WD), jnp.float32),
    compiler_params=pltpu.CompilerParams(
        needs_layout_passes=False, skip_device_barrier=True),
    scratch_types=[
        pltpu.VMEM((_C, _W), jnp.float32),
        pltpu.VMEM((_C, _W), jnp.float32),
        pltpu.VMEM((_C, _W), jnp.float32),
        pltpu.VMEM((_C, _W), jnp.float32),
        pltpu.VMEM((_C * _W,), jnp.float32),
        pltpu.VMEM((_C, _WD), jnp.float32),
        pltpu.VMEM((_C, _WD), jnp.float32),
        pltpu.SemaphoreType.DMA,
        pltpu.SemaphoreType.DMA,
        pltpu.SemaphoreType.DMA,
        pltpu.SemaphoreType.DMA,
    ],
)
def _upsample_sc(x_hbm, out_hbm, a0, b0, a1, b1, row_t, o0, o1,
                 in_sem0, in_sem1, out_sem0, out_sem1):
    cid = lax.axis_index("c")
    sid = lax.axis_index("s")
    wid = sid * 2 + cid
    batch = wid // _WPB
    i_base = (wid % _WPB) * _ROWS_PER_W

    def row_params(i):
        odd = i & 1
        k = i >> 1
        r0 = k - 1 + odd
        w0 = jnp.where(odd == 1, jnp.float32(0.75), jnp.float32(0.25))
        w1 = jnp.where(odd == 1, jnp.float32(0.25), jnp.float32(0.75))
        for si, sr0, sw0, sw1 in _ROW_SPECIALS:
            hit = i == si
            r0 = jnp.where(hit, sr0, r0)
            w0 = jnp.where(hit, jnp.float32(sw0), w0)
            w1 = jnp.where(hit, jnp.float32(sw1), w1)
        return r0, w0, w1

    def fetch(i, a, b, sem):
        r0, _, _ = row_params(i)
        pltpu.async_copy(x_hbm.at[batch, r0], a, sem)
        pltpu.async_copy(x_hbm.at[batch, r0 + 1], b, sem)

    fetch(i_base, a0, b0, in_sem0)
    fetch(i_base + 1, a1, b1, in_sem1)

    jl = lax.iota(jnp.int32, 16)
    zero_v = jl & 0
    fifteen_v = zero_v + 15
    lane0 = jl == 0
    lane15 = jl == 15
    rel_e0 = jnp.maximum((jl - 1) >> 1, 0)
    rel_e1 = (jl + 1) >> 1
    rel_o0 = ((jl + 1) >> 1) + 7
    rel_o1c = jnp.minimum(rel_o0 + 1, 15)
    odd_l = (jl & 1) == 1
    v0_int = jnp.where(odd_l, jnp.float32(0.75), jnp.float32(0.25))
    v1_int = jnp.where(odd_l, jnp.float32(0.25), jnp.float32(0.75))
    v0_b0, v1_b0 = v0_int, v1_int
    v0_b13, v1_b13 = v0_int, v1_int
    idx0_b13 = rel_o0
    for sj, sc0, sv0, sv1 in _COL_SPECIALS:
        if sj < 16:
            hit = jl == sj
            v0_b0 = jnp.where(hit, jnp.float32(sv0), v0_b0)
            v1_b0 = jnp.where(hit, jnp.float32(sv1), v1_b0)
        else:
            hit = jl == (sj - (_WD - 16))
            v0_b13 = jnp.where(hit, jnp.float32(sv0), v0_b13)
            v1_b13 = jnp.where(hit, jnp.float32(sv1), v1_b13)
            idx0_b13 = jnp.where(hit, sc0 - 16 * (_WG - 1), idx0_b13)

    def _dg(vec, idx):
        return jnp.take_along_axis(vec, idx, axis=0)

    _MEM_BLOCKS = (1, 3, 5, 7, 9)

    def do_row(i, m, a, b, o, in_sem, out_sem):
        pltpu.make_async_copy(x_hbm.at[0, 0], a, in_sem).wait()
        pltpu.make_async_copy(x_hbm.at[0, 0], b, in_sem).wait()
        _, w0, w1 = row_params(i)

        @pl.when(m >= 1)
        def _():
            pltpu.make_async_copy(o, out_hbm.at[0, 0], out_sem).wait()

        @plsc.parallel_loop(0, _C, unroll=2)
        def fused(c):
            base = c * _W
            t = []
            for g in range(_WG):
                sl = pl.ds(g * 16, 16)
                tg = w0 * a[c, sl] + w1 * b[c, sl]
                t.append(tg)
                if g < _WG - 1:
                    row_t[pl.ds(base + g * 16, 16)] = tg
            s0 = rel_o0 + base
            for blk in range(_WDG):
                g = blk // 2
                if blk == 0:
                    t0 = _dg(t[0], rel_e0)
                    t1 = _dg(t[0], rel_e1)
                    v0, v1 = v0_b0, v1_b0
                elif blk == _WDG - 1:
                    t0 = _dg(t[_WG - 1], idx0_b13)
                    t1 = _dg(t[_WG - 1], rel_o1c)
                    v0, v1 = v0_b13, v1_b13
                elif blk in _MEM_BLOCKS:
                    t0 = plsc.load_gather(row_t, [s0 + 16 * g])
                    t1 = plsc.load_gather(row_t, [s0 + (16 * g + 1)])
                    v0, v1 = v0_int, v1_int
                elif blk % 2 == 0:
                    t0 = jnp.where(lane0, _dg(t[g - 1], fifteen_v),
                                   _dg(t[g], rel_e0))
                    t1 = _dg(t[g], rel_e1)
                    v0, v1 = v0_int, v1_int
                else:
                    t0 = _dg(t[g], rel_o0)
                    t1 = jnp.where(lane15, _dg(t[g + 1], zero_v),
                                   _dg(t[g], rel_o1c))
                    v0, v1 = v0_int, v1_int
                o[c, pl.ds(blk * 16, 16)] = v0 * t0 + v1 * t1

        pltpu.async_copy(o, out_hbm.at[batch, i], out_sem)

        @pl.when(m < _PAIRS - 1)
        def _():
            fetch(i + 2, a, b, in_sem)

    def per_pair(m, carry):
        i0 = i_base + 2 * m
        do_row(i0, m, a0, b0, o0, in_sem0, out_sem0)
        do_row(i0 + 1, m, a1, b1, o1, in_sem1, out_sem1)
        return carry

    lax.fori_loop(0, _PAIRS, per_pair, 0)
    pltpu.make_async_copy(o0, out_hbm.at[0, 0], out_sem0).wait()
    pltpu.make_async_copy(o1, out_hbm.at[0, 1], out_sem1).wait()


def kernel(inputs):
    x_t = jnp.transpose(inputs, (0, 1, 3, 2))
    out_t = _upsample_sc(x_t)
    return jnp.transpose(out_t, (0, 1, 3, 2))
